# Initial kernel scaffold; baseline (speedup 1.0000x reference)
import jax
import jax.numpy as jnp
from jax.experimental import pallas as pl


def kernel(input_ids, type_ids, voc_table, type_table):
    raise NotImplementedError("write your pallas kernel here")


# speedup vs baseline: 4.3343x; 4.3343x over previous
import functools

import jax
import jax.numpy as jnp
from jax import lax
from jax.experimental import pallas as pl
from jax.experimental.pallas import tpu as pltpu
from jax.experimental.pallas import tpu_sc as plsc

NC = 2
NS = 16
NW = NC * NS
LANES = 16

CHUNK = 512
GATHER = 128


@functools.partial(jax.jit, static_argnames=("n", "h"))
def _embed(ids, tids, voc_table, type_table, *, n, h):
    rw = n // NW
    nch = rw // CHUNK
    mesh = plsc.VectorSubcoreMesh(core_axis_name="c", subcore_axis_name="s")

    @functools.partial(
        pl.kernel,
        mesh=mesh,
        out_type=jax.ShapeDtypeStruct((n, h), jnp.float32),
        compiler_params=pltpu.CompilerParams(use_tc_tiling_on_sc=False),
        scratch_types=[
            pltpu.VMEM((CHUNK,), jnp.int32),
            pltpu.VMEM((CHUNK,), jnp.int32),
            pltpu.VMEM((CHUNK, h), jnp.float32),
            pltpu.VMEM((2 * h,), jnp.float32),
            pltpu.SemaphoreType.DMA,
        ],
    )
    def body(ids_hbm, tids_hbm, voc_hbm, ttab_hbm, out_hbm,
             idx_v, tid_v, rows_v, ttab_v, sem):
        wid = lax.axis_index("c") * NS + lax.axis_index("s")
        base0 = wid * rw
        pltpu.sync_copy(ttab_hbm, ttab_v)
        t0v = [ttab_v[pl.ds(j * LANES, LANES)] for j in range(h // LANES)]
        t1v = [ttab_v[pl.ds(h + j * LANES, LANES)] for j in range(h // LANES)]

        def chunk_body(ci, carry):
            base = base0 + ci * CHUNK
            pltpu.sync_copy(ids_hbm.at[pl.ds(base, CHUNK)], idx_v)
            pltpu.sync_copy(tids_hbm.at[pl.ds(base, CHUNK)], tid_v)
            copies = [
                pltpu.async_copy(
                    voc_hbm.at[idx_v.at[pl.ds(k * GATHER, GATHER)]],
                    rows_v.at[pl.ds(k * GATHER, GATHER)],
                    sem,
                )
                for k in range(CHUNK // GATHER)
            ]
            for cp in copies:
                cp.wait()

            def group_body(g, c2):
                tvec = tid_v[pl.ds(g * LANES, LANES)]
                for i in range(LANES):
                    m = tvec[i] > 0
                    r = g * LANES + i
                    for j in range(h // LANES):
                        sl = pl.ds(j * LANES, LANES)
                        rows_v[r, sl] = rows_v[r, sl] + jnp.where(m, t1v[j], t0v[j])
                return c2

            lax.fori_loop(0, CHUNK // LANES, group_body, 0)
            pltpu.sync_copy(rows_v, out_hbm.at[pl.ds(base, CHUNK)])
            return carry

        lax.fori_loop(0, nch, chunk_body, 0)

    return body(ids, tids, voc_table, type_table)


def kernel(input_ids, type_ids, voc_table, type_table):
    b, l = input_ids.shape
    v, h = voc_table.shape
    n = b * l
    ids = input_ids.reshape(-1).astype(jnp.int32)
    tids = type_ids.reshape(-1).astype(jnp.int32)
    out = _embed(ids, tids, voc_table, type_table.reshape(-1), n=n, h=h)
    return out.reshape(b, l, h)


# speedup vs baseline: 4.8889x; 1.1280x over previous
import functools

import jax
import jax.numpy as jnp
from jax import lax
from jax.experimental import pallas as pl
from jax.experimental.pallas import tpu as pltpu
from jax.experimental.pallas import tpu_sc as plsc

NC = 2
NS = 16
NW = NC * NS
LANES = 16
BLK = 128


@functools.partial(jax.jit, static_argnames=("b", "l", "h"))
def _embed(ids_t, tids_t, voc_table, ttab, *, b, l, h):
    nblk = b // BLK
    blk_per_w = nblk // NW
    hj = h // LANES
    mesh = plsc.VectorSubcoreMesh(core_axis_name="c", subcore_axis_name="s")

    @functools.partial(
        pl.kernel,
        mesh=mesh,
        out_type=jax.ShapeDtypeStruct((l, b, h), jnp.float32),
        compiler_params=pltpu.CompilerParams(use_tc_tiling_on_sc=False),
        scratch_types=[
            pltpu.VMEM((l, BLK), jnp.int32),
            pltpu.VMEM((l, BLK), jnp.int32),
            pltpu.VMEM((BLK, h), jnp.float32),
            pltpu.VMEM((BLK, h), jnp.float32),
            pltpu.VMEM((2 * h,), jnp.float32),
            pltpu.SemaphoreType.DMA,
            pltpu.SemaphoreType.DMA,
        ],
    )
    def body(ids_hbm, tids_hbm, voc_hbm, ttab_hbm, out_hbm,
             idx_v, tid_v, rows_a, rows_b, ttab_v, sem_a, sem_b):
        wid = lax.axis_index("c") * NS + lax.axis_index("s")
        pltpu.sync_copy(ttab_hbm, ttab_v)
        t0v = [ttab_v[pl.ds(j * LANES, LANES)] for j in range(hj)]
        t1v = [ttab_v[pl.ds(h + j * LANES, LANES)] for j in range(hj)]

        def blk_body(q, carry):
            bb = wid * blk_per_w + q
            pltpu.sync_copy(ids_hbm.at[:, pl.ds(bb * BLK, BLK)], idx_v)
            pltpu.sync_copy(tids_hbm.at[:, pl.ds(bb * BLK, BLK)], tid_v)
            pltpu.async_copy(voc_hbm.at[idx_v.at[0]], rows_a, sem_a)

            def step(li, rows, sem, rows_n, sem_n):
                @pl.when(li + 1 < l)
                def _():
                    pltpu.async_copy(voc_hbm.at[idx_v.at[li + 1]], rows_n, sem_n)

                pltpu.make_async_copy(voc_hbm.at[idx_v.at[li]], rows, sem).wait()
                tvecs = [tid_v[li, pl.ds(g * LANES, LANES)]
                         for g in range(BLK // LANES)]
                for bi in range(BLK):
                    m = tvecs[bi // LANES][bi % LANES] > 0
                    for j in range(hj):
                        sl = pl.ds(j * LANES, LANES)
                        rows[bi, sl] = rows[bi, sl] + jnp.where(m, t1v[j], t0v[j])
                pltpu.sync_copy(rows, out_hbm.at[li, pl.ds(bb * BLK, BLK)])

            def pair_body(i2, c2):
                step(2 * i2, rows_a, sem_a, rows_b, sem_b)
                step(2 * i2 + 1, rows_b, sem_b, rows_a, sem_a)
                return c2

            lax.fori_loop(0, l // 2, pair_body, 0)
            return carry

        lax.fori_loop(0, blk_per_w, blk_body, 0)

    return body(ids_t, tids_t, voc_table, ttab)


def kernel(input_ids, type_ids, voc_table, type_table):
    b, l = input_ids.shape
    v, h = voc_table.shape
    th = _embed(input_ids.T.astype(jnp.int32), type_ids.T.astype(jnp.int32),
                voc_table, type_table.reshape(-1), b=b, l=l, h=h)
    return th.transpose(1, 0, 2)
